# initial kernel scaffold (unmeasured)
import jax
import jax.numpy as jnp
from jax import lax
from jax.experimental import pallas as pl
from jax.experimental.pallas import tpu as pltpu


def kernel(
    x,
):
    def body(*refs):
        pass

    out_shape = jax.ShapeDtypeStruct(..., jnp.float32)
    return pl.pallas_call(body, out_shape=out_shape)(...)


# baseline (device time: 143231 ns/iter reference)
import jax
import jax.numpy as jnp
from jax import lax
from jax.experimental import pallas as pl
from jax.experimental.pallas import tpu as pltpu

N_DEV = 32


def kernel(x):
    m, n = x.shape
    c = m // N_DEV

    def body(x_ref, out_ref, rs_send, rs_recv, rs_send_sems, rs_recv_sems,
             ag_send_sems, ag_recv_sems):
        my_pos = lax.axis_index("i")
        left = (my_pos - 1) % N_DEV
        right = (my_pos + 1) % N_DEV

        barrier_sem = pltpu.get_barrier_semaphore()
        for nbr in (left, right):
            pl.semaphore_signal(
                barrier_sem, inc=1,
                device_id=(nbr,), device_id_type=pl.DeviceIdType.MESH,
            )
        pl.semaphore_wait(barrier_sem, 2)

        def chunk(ref, idx):
            return ref.at[pl.ds(idx * c, c), :]

        rs_send[0, :, :] = x_ref[pl.ds(my_pos * c, c), :]
        for s in range(N_DEV - 1):
            rdma = pltpu.make_async_remote_copy(
                src_ref=rs_send.at[s],
                dst_ref=rs_recv.at[s],
                send_sem=rs_send_sems.at[s],
                recv_sem=rs_recv_sems.at[s],
                device_id=(right,),
                device_id_type=pl.DeviceIdType.MESH,
            )
            rdma.start()
            rdma.wait()
            recv_idx = (my_pos - s - 1) % N_DEV
            partial = rs_recv[s, :, :] + x_ref[pl.ds(recv_idx * c, c), :]
            if s < N_DEV - 2:
                rs_send[s + 1, :, :] = partial
            else:
                own_idx = (my_pos + 1) % N_DEV
                out_ref[pl.ds(own_idx * c, c), :] = partial

        for t in range(N_DEV - 1):
            idx = (my_pos + 1 - t) % N_DEV
            rdma = pltpu.make_async_remote_copy(
                src_ref=chunk(out_ref, idx),
                dst_ref=chunk(out_ref, idx),
                send_sem=ag_send_sems.at[t],
                recv_sem=ag_recv_sems.at[t],
                device_id=(right,),
                device_id_type=pl.DeviceIdType.MESH,
            )
            rdma.start()
            rdma.wait()

    return pl.pallas_call(
        body,
        out_shape=jax.ShapeDtypeStruct((m, n), x.dtype),
        in_specs=[pl.BlockSpec(memory_space=pltpu.VMEM)],
        out_specs=pl.BlockSpec(memory_space=pltpu.VMEM),
        scratch_shapes=[
            pltpu.VMEM((N_DEV - 1, c, n), x.dtype),
            pltpu.VMEM((N_DEV - 1, c, n), x.dtype),
            pltpu.SemaphoreType.DMA((N_DEV - 1,)),
            pltpu.SemaphoreType.DMA((N_DEV - 1,)),
            pltpu.SemaphoreType.DMA((N_DEV - 1,)),
            pltpu.SemaphoreType.DMA((N_DEV - 1,)),
        ],
        compiler_params=pltpu.CompilerParams(collective_id=0),
    )(x)


# device time: 33916 ns/iter; 4.2231x vs baseline; 4.2231x over previous
import jax
import jax.numpy as jnp
from jax import lax
from jax.experimental import pallas as pl
from jax.experimental.pallas import tpu as pltpu

N_DEV = 32


def kernel(x):
    m, n = x.shape
    c = m // N_DEV

    def body(x_ref, out_ref, rs_recv, rs_send_sems, rs_recv_sems,
             ag_send_sems, ag_recv_sems):
        my = lax.axis_index("i")

        barrier_sem = pltpu.get_barrier_semaphore()
        for k in range(1, N_DEV):
            pl.semaphore_signal(
                barrier_sem, inc=1,
                device_id=((my + k) % N_DEV,),
                device_id_type=pl.DeviceIdType.MESH,
            )
        pl.semaphore_wait(barrier_sem, N_DEV - 1)

        def rs_desc(p):
            return pltpu.make_async_remote_copy(
                src_ref=x_ref.at[pl.ds(p * c, c), :],
                dst_ref=rs_recv.at[my],
                send_sem=rs_send_sems.at[p],
                recv_sem=rs_recv_sems.at[my],
                device_id=(p,),
                device_id_type=pl.DeviceIdType.MESH,
            )

        def rs_wait_desc(p):
            return pltpu.make_async_remote_copy(
                src_ref=x_ref.at[pl.ds(p * c, c), :],
                dst_ref=rs_recv.at[p],
                send_sem=rs_send_sems.at[p],
                recv_sem=rs_recv_sems.at[p],
                device_id=(p,),
                device_id_type=pl.DeviceIdType.MESH,
            )

        for k in range(1, N_DEV):
            rs_desc((my + k) % N_DEV).start()
        rs_recv[pl.ds(my, 1), :, :] = x_ref[pl.ds(my * c, c), :][None]
        for k in range(1, N_DEV):
            rs_wait_desc((my + k) % N_DEV).wait_recv()
        for k in range(1, N_DEV):
            rs_wait_desc((my + k) % N_DEV).wait_send()

        reduced = jnp.sum(rs_recv[...], axis=0)
        out_ref[pl.ds(my * c, c), :] = reduced

        def ag_desc(p):
            return pltpu.make_async_remote_copy(
                src_ref=out_ref.at[pl.ds(my * c, c), :],
                dst_ref=out_ref.at[pl.ds(my * c, c), :],
                send_sem=ag_send_sems.at[p],
                recv_sem=ag_recv_sems.at[my],
                device_id=(p,),
                device_id_type=pl.DeviceIdType.MESH,
            )

        def ag_wait_desc(p):
            return pltpu.make_async_remote_copy(
                src_ref=out_ref.at[pl.ds(my * c, c), :],
                dst_ref=out_ref.at[pl.ds(p * c, c), :],
                send_sem=ag_send_sems.at[p],
                recv_sem=ag_recv_sems.at[p],
                device_id=(p,),
                device_id_type=pl.DeviceIdType.MESH,
            )

        for k in range(1, N_DEV):
            ag_desc((my + k) % N_DEV).start()
        for k in range(1, N_DEV):
            ag_wait_desc((my + k) % N_DEV).wait_recv()
        for k in range(1, N_DEV):
            ag_wait_desc((my + k) % N_DEV).wait_send()

    return pl.pallas_call(
        body,
        out_shape=jax.ShapeDtypeStruct((m, n), x.dtype),
        in_specs=[pl.BlockSpec(memory_space=pltpu.VMEM)],
        out_specs=pl.BlockSpec(memory_space=pltpu.VMEM),
        scratch_shapes=[
            pltpu.VMEM((N_DEV, c, n), x.dtype),
            pltpu.SemaphoreType.DMA((N_DEV,)),
            pltpu.SemaphoreType.DMA((N_DEV,)),
            pltpu.SemaphoreType.DMA((N_DEV,)),
            pltpu.SemaphoreType.DMA((N_DEV,)),
        ],
        compiler_params=pltpu.CompilerParams(collective_id=0),
    )(x)
